# baseline (speedup 1.0000x reference)
import jax
import jax.numpy as jnp
from jax import lax
from jax.experimental import pallas as pl
from jax.experimental.pallas import tpu as pltpu

H = 1024
F = 2048
E = 8
K = 2
T = 2048

TB = 256
NTB = T // TB
BLK = 256
P = 6144
NB = P // BLK


def _router_body(x_ref, wgt_ref, e1_ref, e2_ref, r1_ref, r2_ref,
                 wa_ref, wb_ref, cnt_ref, carry_ref):
    tb = pl.program_id(0)
    logits = lax.dot_general(
        x_ref[...].astype(jnp.bfloat16), wgt_ref[...].astype(jnp.bfloat16),
        (((1,), (0,)), ((), ())),
        preferred_element_type=jnp.float32)
    lane = lax.broadcasted_iota(jnp.int32, (TB, E), 1)
    big = jnp.int32(E)
    l1 = jnp.max(logits, axis=1, keepdims=True)
    i1 = jnp.min(jnp.where(logits == l1, lane, big), axis=1, keepdims=True)
    masked = jnp.where(lane == i1, -jnp.inf, logits)
    l2 = jnp.max(masked, axis=1, keepdims=True)
    i2 = jnp.min(jnp.where(masked == l2, lane, big), axis=1, keepdims=True)
    wb = 1.0 / (1.0 + jnp.exp(l1 - l2))
    wa = 1.0 - wb

    mask = ((lane == i1) | (lane == i2)).astype(jnp.bfloat16)
    row_i = lax.broadcasted_iota(jnp.int32, (TB, TB), 0)
    col_i = lax.broadcasted_iota(jnp.int32, (TB, TB), 1)
    tri = (col_i < row_i).astype(jnp.bfloat16)
    rank = lax.dot_general(tri, mask, (((1,), (0,)), ((), ())),
                           preferred_element_type=jnp.float32)

    @pl.when(tb == 0)
    def _():
        carry_ref[...] = jnp.zeros_like(carry_ref)

    carry = carry_ref[0:1, 0:E]
    rank = rank + carry
    new_carry = carry + jnp.sum(mask.astype(jnp.float32), axis=0,
                                keepdims=True)
    carry_ref[0:1, 0:E] = new_carry

    e1_ref[...] = i1
    e2_ref[...] = i2
    r1_ref[...] = jnp.sum(jnp.where(lane == i1, rank, 0.0), axis=1,
                          keepdims=True).astype(jnp.int32)
    r2_ref[...] = jnp.sum(jnp.where(lane == i2, rank, 0.0), axis=1,
                          keepdims=True).astype(jnp.int32)
    wa_ref[...] = wa
    wb_ref[...] = wb

    @pl.when(tb == NTB - 1)
    def _():
        cnt_ref[...] = jnp.broadcast_to(new_carry, (E, E))


def _router(x, Wg):
    out_shapes = [
        jax.ShapeDtypeStruct((T, 1), jnp.int32),
        jax.ShapeDtypeStruct((T, 1), jnp.int32),
        jax.ShapeDtypeStruct((T, 1), jnp.int32),
        jax.ShapeDtypeStruct((T, 1), jnp.int32),
        jax.ShapeDtypeStruct((T, 1), jnp.float32),
        jax.ShapeDtypeStruct((T, 1), jnp.float32),
        jax.ShapeDtypeStruct((E, E), jnp.float32),
    ]
    tok_spec = lambda: pl.BlockSpec((TB, 1), lambda tb: (tb, 0))
    return pl.pallas_call(
        _router_body,
        grid=(NTB,),
        in_specs=[
            pl.BlockSpec((TB, H), lambda tb: (tb, 0)),
            pl.BlockSpec((H, E), lambda tb: (0, 0)),
        ],
        out_specs=[
            tok_spec(), tok_spec(), tok_spec(), tok_spec(),
            tok_spec(), tok_spec(),
            pl.BlockSpec((E, E), lambda tb: (0, 0)),
        ],
        out_shape=out_shapes,
        scratch_shapes=[pltpu.VMEM((8, 128), jnp.float32)],
    )(x, Wg.T)


NF = 4
FH = F // NF


def _ffn_body(be_ref, xb_ref, w1_ref, w3_ref, w2_ref,
              p0r_ref, p1r_ref, p0c_ref, p1c_ref, wac_ref, wbc_ref,
              out_ref, xs_s, y_s):
    fh = pl.program_id(0)
    b = pl.program_id(1)
    base = b * BLK
    bsl = pl.ds(base, BLK)

    @pl.when(fh == 0)
    def _():
        rr = lax.broadcasted_iota(jnp.int32, (BLK, T), 0) + base
        gmask = ((p0r_ref[...] == rr)
                 | (p1r_ref[...] == rr)).astype(jnp.bfloat16)
        xs_s[bsl, :] = jnp.dot(gmask, xb_ref[...],
                               preferred_element_type=jnp.float32).astype(
                                   jnp.bfloat16)

    xs = xs_s[bsl, :]
    h = lax.dot_general(xs, w1_ref[0], (((1,), (1,)), ((), ())),
                        preferred_element_type=jnp.float32)
    g = lax.dot_general(xs, w3_ref[0], (((1,), (1,)), ((), ())),
                        preferred_element_type=jnp.float32)
    a = (h * lax.logistic(h) * g).astype(jnp.bfloat16)
    ypart = lax.dot_general(a, w2_ref[0], (((1,), (1,)), ((), ())),
                            preferred_element_type=jnp.float32)

    @pl.when(fh == 0)
    def _():
        y_s[bsl, :] = ypart.astype(jnp.bfloat16)

    @pl.when((fh > 0) & (fh < NF - 1))
    def _():
        y_s[bsl, :] += ypart.astype(jnp.bfloat16)

    @pl.when(fh == NF - 1)
    def _():
        y = y_s[bsl, :] + ypart.astype(jnp.bfloat16)
        rc = lax.broadcasted_iota(jnp.int32, (T, BLK), 1) + base
        sm = (jnp.where(p0c_ref[...] == rc, wac_ref[...], 0.0)
              + jnp.where(p1c_ref[...] == rc, wbc_ref[...], 0.0)).astype(
                  jnp.bfloat16)
        contrib = jnp.dot(sm, y, preferred_element_type=jnp.float32)

        @pl.when(b == 0)
        def _():
            out_ref[...] = contrib

        @pl.when(b > 0)
        def _():
            out_ref[...] += contrib


def _ffn(xb, be, W1, W3, W2, p0r, p1r, p0c, p1c, wac, wbc):
    grid_spec = pltpu.PrefetchScalarGridSpec(
        num_scalar_prefetch=1,
        grid=(NF, NB),
        in_specs=[
            pl.BlockSpec((T, H), lambda fh, b, be: (0, 0)),
            pl.BlockSpec((1, FH, H), lambda fh, b, be: (be[b], fh, 0)),
            pl.BlockSpec((1, FH, H), lambda fh, b, be: (be[b], fh, 0)),
            pl.BlockSpec((1, H, FH), lambda fh, b, be: (be[b], 0, fh)),
            pl.BlockSpec((1, T), lambda fh, b, be: (0, 0)),
            pl.BlockSpec((1, T), lambda fh, b, be: (0, 0)),
            pl.BlockSpec((T, 1), lambda fh, b, be: (0, 0)),
            pl.BlockSpec((T, 1), lambda fh, b, be: (0, 0)),
            pl.BlockSpec((T, 1), lambda fh, b, be: (0, 0)),
            pl.BlockSpec((T, 1), lambda fh, b, be: (0, 0)),
        ],
        out_specs=pl.BlockSpec((T, H), lambda fh, b, be: (0, 0)),
        scratch_shapes=[
            pltpu.VMEM((P, H), jnp.bfloat16),
            pltpu.VMEM((P, H), jnp.bfloat16),
        ],
    )
    return pl.pallas_call(
        _ffn_body,
        grid_spec=grid_spec,
        out_shape=jax.ShapeDtypeStruct((T, H), jnp.float32),
        compiler_params=pltpu.CompilerParams(
            dimension_semantics=("arbitrary", "arbitrary"),
            vmem_limit_bytes=64 * 1024 * 1024),
    )(be, xb, W1, W3, W2, p0r, p1r, p0c, p1c, wac, wbc)


def kernel(x, Wg, W1, W2, W3):
    xb = x.astype(jnp.bfloat16)

    e1, e2, r1, r2, wa, wb, cnt = _router(x, Wg)
    counts = cnt[0].astype(jnp.int32)
    cap = ((counts + (BLK - 1)) // BLK) * BLK
    inc = jnp.cumsum(cap)
    off = (inc - cap).astype(jnp.int32)
    bvec = jnp.arange(NB, dtype=jnp.int32) * BLK
    be = jnp.minimum(
        jnp.sum((inc[None, :] <= bvec[:, None]).astype(jnp.int32), axis=1),
        E - 1).astype(jnp.int32)

    lane = jnp.arange(E, dtype=jnp.int32)[None, :]
    sel1 = (e1 == lane).astype(jnp.int32)
    sel2 = (e2 == lane).astype(jnp.int32)
    p0c = jnp.sum(sel1 * off[None, :], axis=1, keepdims=True) + r1
    p1c = jnp.sum(sel2 * off[None, :], axis=1, keepdims=True) + r2

    return _ffn(xb, be, W1, W3, W2,
                p0c.reshape(1, T), p1c.reshape(1, T),
                p0c, p1c, wa, wb)


# speedup vs baseline: 1.0520x; 1.0520x over previous
import jax
import jax.numpy as jnp
from jax import lax
from jax.experimental import pallas as pl
from jax.experimental.pallas import tpu as pltpu

H = 1024
F = 2048
E = 8
K = 2
T = 2048

TB = 256
NTB = T // TB
BLK = 256
P = 6144
NB = P // BLK


def _router_body(x_ref, wgt_ref, e1_ref, e2_ref, r1_ref, r2_ref,
                 wa_ref, wb_ref, cnt_ref, carry_ref):
    tb = pl.program_id(0)
    logits = lax.dot_general(
        x_ref[...].astype(jnp.bfloat16), wgt_ref[...].astype(jnp.bfloat16),
        (((1,), (0,)), ((), ())),
        preferred_element_type=jnp.float32)
    lane = lax.broadcasted_iota(jnp.int32, (TB, E), 1)
    big = jnp.int32(E)
    l1 = jnp.max(logits, axis=1, keepdims=True)
    i1 = jnp.min(jnp.where(logits == l1, lane, big), axis=1, keepdims=True)
    masked = jnp.where(lane == i1, -jnp.inf, logits)
    l2 = jnp.max(masked, axis=1, keepdims=True)
    i2 = jnp.min(jnp.where(masked == l2, lane, big), axis=1, keepdims=True)
    wb = 1.0 / (1.0 + jnp.exp(l1 - l2))
    wa = 1.0 - wb

    mask = ((lane == i1) | (lane == i2)).astype(jnp.bfloat16)
    row_i = lax.broadcasted_iota(jnp.int32, (TB, TB), 0)
    col_i = lax.broadcasted_iota(jnp.int32, (TB, TB), 1)
    tri = (col_i < row_i).astype(jnp.bfloat16)
    rank = lax.dot_general(tri, mask, (((1,), (0,)), ((), ())),
                           preferred_element_type=jnp.float32)

    @pl.when(tb == 0)
    def _():
        carry_ref[...] = jnp.zeros_like(carry_ref)

    carry = carry_ref[0:1, 0:E]
    rank = rank + carry
    new_carry = carry + jnp.sum(mask.astype(jnp.float32), axis=0,
                                keepdims=True)
    carry_ref[0:1, 0:E] = new_carry

    e1_ref[...] = i1
    e2_ref[...] = i2
    r1_ref[...] = jnp.sum(jnp.where(lane == i1, rank, 0.0), axis=1,
                          keepdims=True).astype(jnp.int32)
    r2_ref[...] = jnp.sum(jnp.where(lane == i2, rank, 0.0), axis=1,
                          keepdims=True).astype(jnp.int32)
    wa_ref[...] = wa
    wb_ref[...] = wb

    @pl.when(tb == NTB - 1)
    def _():
        cnt_ref[...] = jnp.broadcast_to(new_carry, (E, E))


def _router(x, Wg):
    out_shapes = [
        jax.ShapeDtypeStruct((T, 1), jnp.int32),
        jax.ShapeDtypeStruct((T, 1), jnp.int32),
        jax.ShapeDtypeStruct((T, 1), jnp.int32),
        jax.ShapeDtypeStruct((T, 1), jnp.int32),
        jax.ShapeDtypeStruct((T, 1), jnp.float32),
        jax.ShapeDtypeStruct((T, 1), jnp.float32),
        jax.ShapeDtypeStruct((E, E), jnp.float32),
    ]
    tok_spec = lambda: pl.BlockSpec((TB, 1), lambda tb: (tb, 0))
    return pl.pallas_call(
        _router_body,
        grid=(NTB,),
        in_specs=[
            pl.BlockSpec((TB, H), lambda tb: (tb, 0)),
            pl.BlockSpec((H, E), lambda tb: (0, 0)),
        ],
        out_specs=[
            tok_spec(), tok_spec(), tok_spec(), tok_spec(),
            tok_spec(), tok_spec(),
            pl.BlockSpec((E, E), lambda tb: (0, 0)),
        ],
        out_shape=out_shapes,
        scratch_shapes=[pltpu.VMEM((8, 128), jnp.float32)],
    )(x, Wg.T)


FH = F // 2


def _ffn_body(be_ref, xb_ref, w1_ref, w3_ref, w2_ref,
              p0r_ref, p1r_ref, p0c_ref, p1c_ref, wac_ref, wbc_ref,
              out_ref, xs_s, y_s):
    b = pl.program_id(0)
    fh = pl.program_id(1)
    base = b * BLK

    @pl.when(fh == 0)
    def _():
        rr = lax.broadcasted_iota(jnp.int32, (BLK, T), 0) + base
        gmask = ((p0r_ref[...] == rr)
                 | (p1r_ref[...] == rr)).astype(jnp.bfloat16)
        xs_s[...] = jnp.dot(gmask, xb_ref[...],
                            preferred_element_type=jnp.float32).astype(
                                jnp.bfloat16)

    xs = xs_s[...]
    h = lax.dot_general(xs, w1_ref[0], (((1,), (1,)), ((), ())),
                        preferred_element_type=jnp.float32)
    g = lax.dot_general(xs, w3_ref[0], (((1,), (1,)), ((), ())),
                        preferred_element_type=jnp.float32)
    a = (h * lax.logistic(h) * g).astype(jnp.bfloat16)
    ypart = lax.dot_general(a, w2_ref[0], (((1,), (1,)), ((), ())),
                            preferred_element_type=jnp.float32)

    @pl.when(fh == 0)
    def _():
        y_s[...] = ypart

    @pl.when(fh == 1)
    def _():
        y = (y_s[...] + ypart).astype(jnp.bfloat16)
        rc = lax.broadcasted_iota(jnp.int32, (T, BLK), 1) + base
        sm = (jnp.where(p0c_ref[...] == rc, wac_ref[...], 0.0)
              + jnp.where(p1c_ref[...] == rc, wbc_ref[...], 0.0)).astype(
                  jnp.bfloat16)
        contrib = jnp.dot(sm, y, preferred_element_type=jnp.float32)

        @pl.when(b == 0)
        def _():
            out_ref[...] = contrib

        @pl.when(b > 0)
        def _():
            out_ref[...] += contrib


def _ffn(xb, be, W1, W3, W2, p0r, p1r, p0c, p1c, wac, wbc):
    grid_spec = pltpu.PrefetchScalarGridSpec(
        num_scalar_prefetch=1,
        grid=(NB, 2),
        in_specs=[
            pl.BlockSpec((T, H), lambda b, fh, be: (0, 0)),
            pl.BlockSpec((1, FH, H), lambda b, fh, be: (be[b], fh, 0)),
            pl.BlockSpec((1, FH, H), lambda b, fh, be: (be[b], fh, 0)),
            pl.BlockSpec((1, H, FH), lambda b, fh, be: (be[b], 0, fh)),
            pl.BlockSpec((1, T), lambda b, fh, be: (0, 0)),
            pl.BlockSpec((1, T), lambda b, fh, be: (0, 0)),
            pl.BlockSpec((T, 1), lambda b, fh, be: (0, 0)),
            pl.BlockSpec((T, 1), lambda b, fh, be: (0, 0)),
            pl.BlockSpec((T, 1), lambda b, fh, be: (0, 0)),
            pl.BlockSpec((T, 1), lambda b, fh, be: (0, 0)),
        ],
        out_specs=pl.BlockSpec((T, H), lambda b, fh, be: (0, 0)),
        scratch_shapes=[
            pltpu.VMEM((BLK, H), jnp.bfloat16),
            pltpu.VMEM((BLK, H), jnp.float32),
        ],
    )
    return pl.pallas_call(
        _ffn_body,
        grid_spec=grid_spec,
        out_shape=jax.ShapeDtypeStruct((T, H), jnp.float32),
        compiler_params=pltpu.CompilerParams(
            dimension_semantics=("arbitrary", "arbitrary")),
    )(be, xb, W1, W3, W2, p0r, p1r, p0c, p1c, wac, wbc)


def kernel(x, Wg, W1, W2, W3):
    xb = x.astype(jnp.bfloat16)

    e1, e2, r1, r2, wa, wb, cnt = _router(x, Wg)
    counts = cnt[0].astype(jnp.int32)
    cap = ((counts + (BLK - 1)) // BLK) * BLK
    inc = jnp.cumsum(cap)
    off = (inc - cap).astype(jnp.int32)
    bvec = jnp.arange(NB, dtype=jnp.int32) * BLK
    be = jnp.minimum(
        jnp.sum((inc[None, :] <= bvec[:, None]).astype(jnp.int32), axis=1),
        E - 1).astype(jnp.int32)

    lane = jnp.arange(E, dtype=jnp.int32)[None, :]
    sel1 = (e1 == lane).astype(jnp.int32)
    sel2 = (e2 == lane).astype(jnp.int32)
    p0c = jnp.sum(sel1 * off[None, :], axis=1, keepdims=True) + r1
    p1c = jnp.sum(sel2 * off[None, :], axis=1, keepdims=True) + r2

    return _ffn(xb, be, W1, W3, W2,
                p0c.reshape(1, T), p1c.reshape(1, T),
                p0c, p1c, wa, wb)
